# Initial kernel scaffold; baseline (speedup 1.0000x reference)
import functools

import jax
import jax.numpy as jnp
from jax import lax
from jax.experimental import pallas as pl
from jax.experimental.pallas import tpu as pltpu
from jax.experimental.pallas import tpu_sc as plsc

_NUM_CORES = 2
_NUM_SUBCORES = 16
_LANES = 16
_NUM_WORKERS = _NUM_CORES * _NUM_SUBCORES

_CHUNK_ROWS = 64


def _hist_body(idx_hbm, counts_hbm, idx_v, hist_v):
    B, S = idx_hbm.shape
    V = counts_hbm.shape[1]
    rows_per_w = B // _NUM_WORKERS
    wid = lax.axis_index("s") * _NUM_CORES + lax.axis_index("c")
    base = wid * rows_per_w

    lane = jnp.arange(_LANES, dtype=jnp.int32)
    ones = jnp.full((_LANES,), 1.0, dtype=jnp.float32)
    zeros = jnp.zeros((_LANES,), dtype=jnp.float32)
    groups = _CHUNK_ROWS // _LANES
    row_ids = [jnp.int32(g * _LANES) + lane for g in range(groups)]

    for chunk in range(rows_per_w // _CHUNK_ROWS):
        rb = base + chunk * _CHUNK_ROWS
        pltpu.sync_copy(idx_hbm.at[pl.ds(rb, _CHUNK_ROWS), :], idx_v)

        def _zero(i, _):
            for j in range(V // _LANES):
                hist_v[i, pl.ds(j * _LANES, _LANES)] = zeros
            return 0

        lax.fori_loop(0, _CHUNK_ROWS, _zero, 0)

        def _step(s, _):
            sv = jnp.full((_LANES,), s, dtype=jnp.int32)
            for g in range(groups):
                v = plsc.load_gather(idx_v, [row_ids[g], sv])
                plsc.addupdate_scatter(hist_v, [row_ids[g], v], ones)
            return 0

        lax.fori_loop(0, S, _step, 0)
        pltpu.sync_copy(hist_v, counts_hbm.at[pl.ds(rb, _CHUNK_ROWS), :])


def _histogram(char_indices, vocab):
    B, S = char_indices.shape
    mesh = plsc.VectorSubcoreMesh(
        core_axis_name="c", subcore_axis_name="s",
        num_cores=_NUM_CORES, num_subcores=_NUM_SUBCORES)
    return pl.kernel(
        _hist_body,
        out_type=jax.ShapeDtypeStruct((B, vocab), jnp.float32),
        mesh=mesh,
        scratch_types=[
            pltpu.VMEM((_CHUNK_ROWS, S), jnp.int32),
            pltpu.VMEM((_CHUNK_ROWS, vocab), jnp.float32),
        ],
    )(char_indices)


def _mm_body(seq_len, counts_ref, emb_ref, pos_ref, out_ref):
    inv = 1.0 / seq_len
    pos_bias = jnp.sum(pos_ref[...], axis=0, keepdims=True) * inv
    acc = jnp.dot(counts_ref[...], emb_ref[...],
                  preferred_element_type=jnp.float32)
    out_ref[...] = acc * inv + pos_bias


def _combine(counts, embedding, pos_used):
    B, V = counts.shape
    D = embedding.shape[1]
    S = pos_used.shape[0]
    TB = 2048
    grid = (B // TB,)
    return pl.pallas_call(
        functools.partial(_mm_body, float(S)),
        grid=grid,
        in_specs=[
            pl.BlockSpec((TB, V), lambda i: (i, 0)),
            pl.BlockSpec((V, D), lambda i: (0, 0)),
            pl.BlockSpec((S, D), lambda i: (0, 0)),
        ],
        out_specs=pl.BlockSpec((TB, D), lambda i: (i, 0)),
        out_shape=jax.ShapeDtypeStruct((B, D), jnp.float32),
    )(counts, embedding, pos_used)


def kernel(char_indices, embedding, pos_embedding):
    B, S = char_indices.shape
    V = embedding.shape[0]
    counts = _histogram(char_indices.astype(jnp.int32), V)
    return _combine(counts, embedding, pos_embedding[:S])


# speedup vs baseline: 92.4291x; 92.4291x over previous
import functools

import jax
import jax.numpy as jnp
from jax import lax
from jax.experimental import pallas as pl
from jax.experimental.pallas import tpu as pltpu
from jax.experimental.pallas import tpu_sc as plsc

_NUM_CORES = 2
_NUM_SUBCORES = 16
_LANES = 16
_NUM_WORKERS = _NUM_CORES * _NUM_SUBCORES

_CHUNK_ROWS = 64


def _hist_body(idx_hbm, counts_hbm, idx_v, hist_v):
    B, S = idx_hbm.shape
    V = counts_hbm.shape[1]
    rows_per_w = B // _NUM_WORKERS
    wid = lax.axis_index("s") * _NUM_CORES + lax.axis_index("c")
    base = wid * rows_per_w

    lane = jnp.arange(_LANES, dtype=jnp.int32)
    ones = jnp.full((_LANES,), 1.0, dtype=jnp.float32)
    zeros = jnp.zeros((_LANES,), dtype=jnp.float32)
    groups = _CHUNK_ROWS // _LANES
    row_ids = [jnp.int32(g * _LANES) + lane for g in range(groups)]

    for chunk in range(rows_per_w // _CHUNK_ROWS):
        rb = base + chunk * _CHUNK_ROWS
        pltpu.sync_copy(idx_hbm.at[pl.ds(rb, _CHUNK_ROWS), :], idx_v)

        def _zero(i, _):
            for j in range(V // _LANES):
                hist_v[i, pl.ds(j * _LANES, _LANES)] = zeros
            return 0

        lax.fori_loop(0, _CHUNK_ROWS, _zero, 0)

        def _step(s, _):
            sv = jnp.full((_LANES,), s, dtype=jnp.int32)
            for g in range(groups):
                v = plsc.load_gather(idx_v, [row_ids[g], sv])
                plsc.addupdate_scatter(hist_v, [row_ids[g], v], ones)
            return 0

        lax.fori_loop(0, S, _step, 0)
        pltpu.sync_copy(hist_v, counts_hbm.at[pl.ds(rb, _CHUNK_ROWS), :])


def _histogram(char_indices, vocab):
    B, S = char_indices.shape
    mesh = plsc.VectorSubcoreMesh(
        core_axis_name="c", subcore_axis_name="s",
        num_cores=_NUM_CORES, num_subcores=_NUM_SUBCORES)
    return pl.kernel(
        _hist_body,
        out_type=jax.ShapeDtypeStruct((B, vocab), jnp.float32),
        mesh=mesh,
        compiler_params=pltpu.CompilerParams(
            use_tc_tiling_on_sc=False, needs_layout_passes=False),
        scratch_types=[
            pltpu.VMEM((_CHUNK_ROWS, S), jnp.int32),
            pltpu.VMEM((_CHUNK_ROWS, vocab), jnp.float32),
        ],
    )(char_indices)


def _mm_body(seq_len, counts_ref, emb_ref, pos_ref, out_ref):
    inv = 1.0 / seq_len
    pos_bias = jnp.sum(pos_ref[...], axis=0, keepdims=True) * inv
    acc = jnp.dot(counts_ref[...], emb_ref[...],
                  preferred_element_type=jnp.float32)
    out_ref[...] = acc * inv + pos_bias


def _combine(counts, embedding, pos_used):
    B, V = counts.shape
    D = embedding.shape[1]
    S = pos_used.shape[0]
    TB = 2048
    grid = (B // TB,)
    return pl.pallas_call(
        functools.partial(_mm_body, float(S)),
        grid=grid,
        in_specs=[
            pl.BlockSpec((TB, V), lambda i: (i, 0)),
            pl.BlockSpec((V, D), lambda i: (0, 0)),
            pl.BlockSpec((S, D), lambda i: (0, 0)),
        ],
        out_specs=pl.BlockSpec((TB, D), lambda i: (i, 0)),
        out_shape=jax.ShapeDtypeStruct((B, D), jnp.float32),
    )(counts, embedding, pos_used)


def kernel(char_indices, embedding, pos_embedding):
    B, S = char_indices.shape
    V = embedding.shape[0]
    counts = _histogram(char_indices.astype(jnp.int32), V)
    return _combine(counts, embedding, pos_embedding[:S])


# speedup vs baseline: 147.7898x; 1.5990x over previous
import functools

import jax
import jax.numpy as jnp
from jax import lax
from jax.experimental import pallas as pl
from jax.experimental.pallas import tpu as pltpu
from jax.experimental.pallas import tpu_sc as plsc

_NUM_CORES = 2
_NUM_SUBCORES = 16
_LANES = 16
_NUM_WORKERS = _NUM_CORES * _NUM_SUBCORES

_CHUNK_ROWS = 128
_S_UNROLL = 2


def _hist_body(idx_hbm, counts_hbm, idx_v0, idx_v1, hist_v0, hist_v1,
               si0, si1, so0, so1):
    B, S = idx_hbm.shape
    V = counts_hbm.shape[1]
    rows_per_w = B // _NUM_WORKERS
    n_chunks = rows_per_w // _CHUNK_ROWS
    wid = lax.axis_index("s") * _NUM_CORES + lax.axis_index("c")
    base = wid * rows_per_w

    lane = jnp.arange(_LANES, dtype=jnp.int32)
    ones = jnp.full((_LANES,), 1.0, dtype=jnp.float32)
    zeros = jnp.zeros((_LANES,), dtype=jnp.float32)
    groups = _CHUNK_ROWS // _LANES
    row_ids = [jnp.int32(g * _LANES) + lane for g in range(groups)]

    idx_bufs = (idx_v0, idx_v1)
    hist_bufs = (hist_v0, hist_v1)
    in_sems = (si0, si1)
    out_sems = (so0, so1)
    in_h = [None] * n_chunks
    out_h = [None] * n_chunks

    in_h[0] = pltpu.async_copy(
        idx_hbm.at[pl.ds(base, _CHUNK_ROWS), :], idx_bufs[0], in_sems[0])

    for chunk in range(n_chunks):
        b = chunk % 2
        if chunk + 1 < n_chunks:
            nb = (chunk + 1) % 2
            rb_next = base + (chunk + 1) * _CHUNK_ROWS
            in_h[chunk + 1] = pltpu.async_copy(
                idx_hbm.at[pl.ds(rb_next, _CHUNK_ROWS), :],
                idx_bufs[nb], in_sems[nb])
        in_h[chunk].wait()
        if chunk >= 2:
            out_h[chunk - 2].wait()
        hist_v = hist_bufs[b]
        idx_v = idx_bufs[b]

        @functools.partial(plsc.parallel_loop, 0, _CHUNK_ROWS)
        def _zero(i):
            for j in range(V // _LANES):
                hist_v[i, pl.ds(j * _LANES, _LANES)] = zeros

        def _step(_, sv):
            vals = []
            for u in range(_S_UNROLL):
                svu = sv + u if u else sv
                for g in range(groups):
                    vals.append(
                        (g, plsc.load_gather(idx_v, [row_ids[g], svu])))
            for g, v in vals:
                plsc.addupdate_scatter(hist_v, [row_ids[g], v], ones)
            return sv + _S_UNROLL

        lax.fori_loop(0, S // _S_UNROLL, _step,
                      jnp.zeros((_LANES,), jnp.int32))
        for s_tail in range(S - S % _S_UNROLL, S):
            svt = jnp.full((_LANES,), s_tail, dtype=jnp.int32)
            vals = []
            for g in range(groups):
                vals.append(
                    (g, plsc.load_gather(idx_v, [row_ids[g], svt])))
            for g, v in vals:
                plsc.addupdate_scatter(hist_v, [row_ids[g], v], ones)

        rb = base + chunk * _CHUNK_ROWS
        out_h[chunk] = pltpu.async_copy(
            hist_v, counts_hbm.at[pl.ds(rb, _CHUNK_ROWS), :], out_sems[b])
    for chunk in range(max(0, n_chunks - 2), n_chunks):
        out_h[chunk].wait()


def _histogram(char_indices, vocab):
    B, S = char_indices.shape
    mesh = plsc.VectorSubcoreMesh(
        core_axis_name="c", subcore_axis_name="s",
        num_cores=_NUM_CORES, num_subcores=_NUM_SUBCORES)
    return pl.kernel(
        _hist_body,
        out_type=jax.ShapeDtypeStruct((B, vocab), jnp.float32),
        mesh=mesh,
        compiler_params=pltpu.CompilerParams(
            use_tc_tiling_on_sc=False, needs_layout_passes=False),
        scratch_types=[
            pltpu.VMEM((_CHUNK_ROWS, S), jnp.int32),
            pltpu.VMEM((_CHUNK_ROWS, S), jnp.int32),
            pltpu.VMEM((_CHUNK_ROWS, vocab), jnp.float32),
            pltpu.VMEM((_CHUNK_ROWS, vocab), jnp.float32),
            pltpu.SemaphoreType.DMA,
            pltpu.SemaphoreType.DMA,
            pltpu.SemaphoreType.DMA,
            pltpu.SemaphoreType.DMA,
        ],
    )(char_indices)


def _mm_body(seq_len, counts_ref, emb_ref, pos_ref, out_ref):
    inv = 1.0 / seq_len
    pos_bias = jnp.sum(pos_ref[...], axis=0, keepdims=True) * inv
    acc = jnp.dot(counts_ref[...], emb_ref[...],
                  preferred_element_type=jnp.float32)
    out_ref[...] = acc * inv + pos_bias


def _combine(counts, embedding, pos_used):
    B, V = counts.shape
    D = embedding.shape[1]
    S = pos_used.shape[0]
    TB = 2048
    grid = (B // TB,)
    return pl.pallas_call(
        functools.partial(_mm_body, float(S)),
        grid=grid,
        in_specs=[
            pl.BlockSpec((TB, V), lambda i: (i, 0)),
            pl.BlockSpec((V, D), lambda i: (0, 0)),
            pl.BlockSpec((S, D), lambda i: (0, 0)),
        ],
        out_specs=pl.BlockSpec((TB, D), lambda i: (i, 0)),
        out_shape=jax.ShapeDtypeStruct((B, D), jnp.float32),
    )(counts, embedding, pos_used)


def kernel(char_indices, embedding, pos_embedding):
    B, S = char_indices.shape
    V = embedding.shape[0]
    counts = _histogram(char_indices.astype(jnp.int32), V)
    return _combine(counts, embedding, pos_embedding[:S])


# speedup vs baseline: 287.6011x; 1.9460x over previous
import functools

import jax
import jax.numpy as jnp
from jax import lax
from jax.experimental import pallas as pl
from jax.experimental.pallas import tpu as pltpu
from jax.experimental.pallas import tpu_sc as plsc

_NUM_CORES = 2
_NUM_SUBCORES = 16
_LANES = 16
_NUM_WORKERS = _NUM_CORES * _NUM_SUBCORES

_S_UNROLL = 4


def _hist_body(idx4_hbm, counts_hbm, idx_v0, idx_v1, hist_v0, hist_v1,
               si0, si1, so0, so1):
    SB, BB, s_in, b_in = idx4_hbm.shape
    V = counts_hbm.shape[1]
    blocks_per_w = BB // _NUM_WORKERS
    wid = lax.axis_index("s") * _NUM_CORES + lax.axis_index("c")
    base_blk = wid * blocks_per_w

    lane = jnp.arange(_LANES, dtype=jnp.int32)
    ones = jnp.full((_LANES,), 1.0, dtype=jnp.float32)
    zeros = jnp.zeros((_LANES,), dtype=jnp.float32)
    groups = b_in // _LANES
    row_ids = [jnp.int32(g * _LANES) + lane for g in range(groups)]

    idx_bufs = (idx_v0, idx_v1)
    hist_bufs = (hist_v0, hist_v1)
    in_sems = (si0, si1)
    out_sems = (so0, so1)
    in_h = [None] * blocks_per_w
    out_h = [None] * blocks_per_w

    def _fire_in(blk, buf, sem):
        return [pltpu.async_copy(
            idx4_hbm.at[sb_, blk, :, :],
            buf.at[pl.ds(sb_ * s_in, s_in), :], sem) for sb_ in range(SB)]

    in_h[0] = _fire_in(base_blk, idx_bufs[0], in_sems[0])

    for chunk in range(blocks_per_w):
        b = chunk % 2
        if chunk + 1 < blocks_per_w:
            nb = (chunk + 1) % 2
            in_h[chunk + 1] = _fire_in(
                base_blk + chunk + 1, idx_bufs[nb], in_sems[nb])
        for h in in_h[chunk]:
            h.wait()
        if chunk >= 2:
            out_h[chunk - 2].wait()
        hist_v = hist_bufs[b]
        idx_v = idx_bufs[b]

        def _zero(i, _):
            r0 = i * 8
            for r in range(8):
                for j in range(V // _LANES):
                    hist_v[r0 + r, pl.ds(j * _LANES, _LANES)] = zeros
                if V % _LANES:
                    hist_v[r0 + r, pl.ds(V - _LANES, _LANES)] = zeros
            return 0

        lax.fori_loop(0, b_in // 8, _zero, 0)

        def _step(sb, _):
            s0 = sb * s_in
            for sp in range(s_in // _S_UNROLL):
                vals = []
                for u in range(_S_UNROLL):
                    si = sp * _S_UNROLL + u
                    for g in range(groups):
                        vals.append(
                            (g, idx_v[s0 + si, pl.ds(g * _LANES, _LANES)]))
                for g, v in vals:
                    plsc.addupdate_scatter(hist_v, [row_ids[g], v], ones)
            return 0

        lax.fori_loop(0, SB, _step, 0)

        cb = (base_blk + chunk) * b_in
        out_h[chunk] = pltpu.async_copy(
            hist_v, counts_hbm.at[pl.ds(cb, b_in), :], out_sems[b])
    for chunk in range(max(0, blocks_per_w - 2), blocks_per_w):
        out_h[chunk].wait()


def _histogram(idx4, vocab):
    SB, BB, s_in, b_in = idx4.shape
    B = BB * b_in
    mesh = plsc.VectorSubcoreMesh(
        core_axis_name="c", subcore_axis_name="s",
        num_cores=_NUM_CORES, num_subcores=_NUM_SUBCORES)
    return pl.kernel(
        _hist_body,
        out_type=jax.ShapeDtypeStruct((B, vocab), jnp.float32),
        mesh=mesh,
        compiler_params=pltpu.CompilerParams(
            use_tc_tiling_on_sc=False, needs_layout_passes=False),
        scratch_types=[
            pltpu.VMEM((SB * s_in, b_in), jnp.int32),
            pltpu.VMEM((SB * s_in, b_in), jnp.int32),
            pltpu.VMEM((b_in, vocab), jnp.float32),
            pltpu.VMEM((b_in, vocab), jnp.float32),
            pltpu.SemaphoreType.DMA,
            pltpu.SemaphoreType.DMA,
            pltpu.SemaphoreType.DMA,
            pltpu.SemaphoreType.DMA,
        ],
    )(idx4)


def _mm_body(seq_len, counts_ref, emb_ref, pos_ref, out_ref):
    inv = 1.0 / seq_len
    pos_bias = jnp.sum(pos_ref[...], axis=0, keepdims=True) * inv
    acc = jnp.dot(counts_ref[...], emb_ref[...],
                  preferred_element_type=jnp.float32)
    out_ref[...] = (acc * inv + pos_bias).T


def _combine(counts, embedding, pos_used):
    B, V = counts.shape
    D = embedding.shape[1]
    S = pos_used.shape[0]
    TB = 4096
    grid = (B // TB,)
    return pl.pallas_call(
        functools.partial(_mm_body, float(S)),
        grid=grid,
        in_specs=[
            pl.BlockSpec((TB, V), lambda i: (i, 0)),
            pl.BlockSpec((V, D), lambda i: (0, 0)),
            pl.BlockSpec((S, D), lambda i: (0, 0)),
        ],
        out_specs=pl.BlockSpec((D, TB), lambda i: (0, i)),
        out_shape=jax.ShapeDtypeStruct((D, B), jnp.float32),
        compiler_params=pltpu.CompilerParams(
            dimension_semantics=("parallel",)),
    )(counts, embedding, pos_used)


def kernel(char_indices, embedding, pos_embedding):
    B, S = char_indices.shape
    V = embedding.shape[0]
    idx4 = (char_indices.astype(jnp.int32).T
            .reshape(S // 8, 8, B // 128, 128).swapaxes(1, 2))
    counts = _histogram(idx4, V)
    return _combine(counts, embedding, pos_embedding[:S]).T
